# baseline (speedup 1.0000x reference)
import jax
import jax.numpy as jnp
import numpy as np
from jax.experimental import pallas as pl
from jax.experimental.pallas import tpu as pltpu

MAX_STEPS_ = 1000
ROWS_PER_BLOCK = 256
ROW_LEN = 2048
CHUNK_R = 8
CHUNK_C = 512

_U32 = jnp.uint32
_KS1 = 42
_KS2 = 0x1BD11BDA ^ 42

_G = [1.2543749809265137, -0.023982059210538864, 0.45813021063804626,
      -0.28965041041374207, 0.33574575185775757, -0.1841685026884079,
      0.04992347210645676, -0.006709587294608355, 0.0003595015441533178]

_UNIF_LO = np.nextafter(np.float32(-1.0), np.float32(0.0))
_UNIF_SPAN = np.float32(np.float32(1.0) - _UNIF_LO)
_UNIF_OFF = np.float32(_UNIF_LO - _UNIF_SPAN)


def _rotl(x, r):
    return jax.lax.shift_left(x, _U32(r)) | jax.lax.shift_right_logical(x, _U32(32 - r))


def _threefry_bits(x1):
    x0 = x1
    x1 = x0 ^ _rotl(x1, 13)
    for r in (15, 26, 6):
        x0 = x0 + x1
        x1 = x0 ^ _rotl(x1, r)
    x0 = x0 + _U32(_KS1)
    x1 = x1 + _U32(_KS2 + 1)
    for g, rots in ((1, (17, 29, 16, 24)), (2, (13, 15, 26, 6)),
                    (3, (17, 29, 16, 24)), (4, (13, 15, 26, 6))):
        for r in rots:
            x0 = x0 + x1
            x1 = x0 ^ _rotl(x1, r)
        ks = (0, _KS1, _KS2)
        x0 = x0 + _U32(ks[(g + 1) % 3])
        x1 = x1 + _U32((ks[(g + 2) % 3] + g + 1) % (1 << 32))
    return x0 ^ x1


def _bits_to_normal(bits):
    f = jax.lax.bitcast_convert_type(
        jax.lax.shift_right_logical(bits, _U32(9)) | _U32(0x3F800000), jnp.float32)
    u = f * _UNIF_SPAN + _UNIF_OFF
    u = jnp.clip(u, _UNIF_LO, -_UNIF_LO)
    s = 1.0 - u * u
    sq = jnp.sqrt(-jnp.log(s))
    p = jnp.float32(_G[-1])
    for c in _G[-2::-1]:
        p = jnp.float32(c) + p * sq
    return p * u


def _block_body(x_ref, c_ref, o_ref):
    i = pl.program_id(0)
    rows, cols = x_ref.shape
    nc = cols // CHUNK_C
    nchunks = (rows // CHUNK_R) * nc
    iota = (jax.lax.broadcasted_iota(_U32, (CHUNK_R, CHUNK_C), 0) * _U32(cols)
            + jax.lax.broadcasted_iota(_U32, (CHUNK_R, CHUNK_C), 1)
            + _U32(_KS1))
    block_base = i * rows * cols

    def body(k, carry):
        r = (k // nc) * CHUNK_R
        c = (k % nc) * CHUNK_C
        base = (block_base + r * cols + c).astype(_U32)
        noise = _bits_to_normal(_threefry_bits(iota + base))
        x = x_ref[pl.ds(r, CHUNK_R), pl.ds(c, CHUNK_C)]
        coef = c_ref[pl.ds(r, CHUNK_R), :]
        o_ref[pl.ds(r, CHUNK_R), pl.ds(c, CHUNK_C)] = x + coef * (noise - x)
        return carry

    jax.lax.fori_loop(0, nchunks, body, 0)


def kernel(X, steps, mask):
    b, s, d = X.shape
    n_rows = b * s
    ni = 1.0 - jnp.cos(jnp.pi * (1.0 - steps.astype(X.dtype) / MAX_STEPS_) / 2.0)
    coef = jnp.where(mask, (1.0 - ni)[:, None], 0.0).astype(X.dtype)
    coef = coef.reshape(n_rows, 1)
    x2 = X.reshape(n_rows, d)
    grid = n_rows // ROWS_PER_BLOCK
    out = pl.pallas_call(
        _block_body,
        grid=(grid,),
        in_specs=[
            pl.BlockSpec((ROWS_PER_BLOCK, d), lambda i: (i, 0)),
            pl.BlockSpec((ROWS_PER_BLOCK, 1), lambda i: (i, 0)),
        ],
        out_specs=pl.BlockSpec((ROWS_PER_BLOCK, d), lambda i: (i, 0)),
        out_shape=jax.ShapeDtypeStruct((n_rows, d), X.dtype),
        compiler_params=pltpu.CompilerParams(
            dimension_semantics=("parallel",)),
    )(x2, coef)
    return out.reshape(b, s, d)


# speedup vs baseline: 1.7483x; 1.7483x over previous
import jax
import jax.numpy as jnp
import numpy as np
from jax.experimental import pallas as pl
from jax.experimental.pallas import tpu as pltpu

MAX_STEPS_ = 1000
ROWS_PER_BLOCK = 256
ROW_LEN = 2048
CHUNK_R = 8
CHUNK_C = 2048

_U32 = jnp.uint32
_KS1 = 42
_KS2 = 0x1BD11BDA ^ 42

_G = [1.2543749809265137, -0.023982059210538864, 0.45813021063804626,
      -0.28965041041374207, 0.33574575185775757, -0.1841685026884079,
      0.04992347210645676, -0.006709587294608355, 0.0003595015441533178]

_UNIF_LO = np.nextafter(np.float32(-1.0), np.float32(0.0))
_UNIF_SPAN = np.float32(np.float32(1.0) - _UNIF_LO)
_UNIF_OFF = np.float32(_UNIF_LO - _UNIF_SPAN)


def _rotl(x, r):
    return jax.lax.shift_left(x, _U32(r)) | jax.lax.shift_right_logical(x, _U32(32 - r))


def _threefry_bits(x1):
    x0 = x1
    x1 = x0 ^ _rotl(x1, 13)
    for r in (15, 26, 6):
        x0 = x0 + x1
        x1 = x0 ^ _rotl(x1, r)
    x0 = x0 + _U32(_KS1)
    x1 = x1 + _U32(_KS2 + 1)
    for g, rots in ((1, (17, 29, 16, 24)), (2, (13, 15, 26, 6)),
                    (3, (17, 29, 16, 24)), (4, (13, 15, 26, 6))):
        for r in rots:
            x0 = x0 + x1
            x1 = x0 ^ _rotl(x1, r)
        ks = (0, _KS1, _KS2)
        x0 = x0 + _U32(ks[(g + 1) % 3])
        x1 = x1 + _U32((ks[(g + 2) % 3] + g + 1) % (1 << 32))
    return x0 ^ x1


def _bits_to_normal(bits):
    f = jax.lax.bitcast_convert_type(
        jax.lax.shift_right_logical(bits, _U32(9)) | _U32(0x3F800000), jnp.float32)
    u = f * _UNIF_SPAN + _UNIF_OFF
    u = jnp.clip(u, _UNIF_LO, -_UNIF_LO)
    s = 1.0 - u * u
    sq = jnp.sqrt(-jnp.log(s))
    p = jnp.float32(_G[-1])
    for c in _G[-2::-1]:
        p = jnp.float32(c) + p * sq
    return p * u


def _block_body(x_ref, c_ref, o_ref):
    i = pl.program_id(0)
    rows, cols = x_ref.shape
    nc = cols // CHUNK_C
    nchunks = (rows // CHUNK_R) * nc
    iota = (jax.lax.broadcasted_iota(_U32, (CHUNK_R, CHUNK_C), 0) * _U32(cols)
            + jax.lax.broadcasted_iota(_U32, (CHUNK_R, CHUNK_C), 1)
            + _U32(_KS1))
    block_base = i * rows * cols

    def body(k, carry):
        r = (k // nc) * CHUNK_R
        c = (k % nc) * CHUNK_C
        base = (block_base + r * cols + c).astype(_U32)
        noise = _bits_to_normal(_threefry_bits(iota + base))
        x = x_ref[pl.ds(r, CHUNK_R), pl.ds(c, CHUNK_C)]
        coef = c_ref[pl.ds(r, CHUNK_R), :]
        o_ref[pl.ds(r, CHUNK_R), pl.ds(c, CHUNK_C)] = x + coef * (noise - x)
        return carry

    jax.lax.fori_loop(0, nchunks, body, 0)


def kernel(X, steps, mask):
    b, s, d = X.shape
    n_rows = b * s
    ni = 1.0 - jnp.cos(jnp.pi * (1.0 - steps.astype(X.dtype) / MAX_STEPS_) / 2.0)
    coef = jnp.where(mask, (1.0 - ni)[:, None], 0.0).astype(X.dtype)
    coef = coef.reshape(n_rows, 1)
    x2 = X.reshape(n_rows, d)
    grid = n_rows // ROWS_PER_BLOCK
    out = pl.pallas_call(
        _block_body,
        grid=(grid,),
        in_specs=[
            pl.BlockSpec((ROWS_PER_BLOCK, d), lambda i: (i, 0)),
            pl.BlockSpec((ROWS_PER_BLOCK, 1), lambda i: (i, 0)),
        ],
        out_specs=pl.BlockSpec((ROWS_PER_BLOCK, d), lambda i: (i, 0)),
        out_shape=jax.ShapeDtypeStruct((n_rows, d), X.dtype),
        compiler_params=pltpu.CompilerParams(
            dimension_semantics=("parallel",)),
    )(x2, coef)
    return out.reshape(b, s, d)
